# Initial kernel scaffold; baseline (speedup 1.0000x reference)
import jax
import jax.numpy as jnp
from jax.experimental import pallas as pl


def kernel(img, w):
    raise NotImplementedError("write your pallas kernel here")


# speedup vs baseline: 13.3651x; 13.3651x over previous
import functools

import jax
import jax.numpy as jnp
from jax import lax
from jax.experimental import pallas as pl
from jax.experimental.pallas import tpu as pltpu
from jax.experimental.pallas import tpu_sc as plsc

_N, _C, _H, _W = 16, 3, 512, 512
_P = _H * _W
_TOPK = int(_P * 0.05)
_NB = 4096
_LANES = 16
_HIST_WORDS = _NB * _LANES
_CH = 16384
_HALF = _P // 2


def _dark_body(img_ref, out_ref):
    x = img_ref[0]
    m = jnp.minimum(jnp.minimum(x[0], x[1]), x[2])

    zh = jnp.zeros((_H, 8), jnp.float32)
    v = jnp.concatenate([zh, m, zh], axis=1)
    p1 = v[:, :525] + v[:, 1:526]
    p2 = p1[:, :521] + p1[:, 2:523]
    p3 = p2[:, :513] + p2[:, 4:517]
    s15 = p3[:, 1:513] + p2[:, 9:521] + p1[:, 13:525] + v[:, 15:527]

    zv = jnp.zeros((8, _W), jnp.float32)
    u = jnp.concatenate([zv, s15, zv], axis=0)
    q1 = u[:525] + u[1:526]
    q2 = q1[:521] + q1[2:523]
    q3 = q2[:513] + q2[4:517]
    dk = (q3[1:513] + q2[9:521] + q1[13:525] + u[15:527]) * (1.0 / 225.0)
    out_ref[0] = dk


@functools.cache
def _hist_call():
    mesh = plsc.VectorSubcoreMesh(
        core_axis_name="c", subcore_axis_name="s",
        num_cores=2, num_subcores=16)
    return pl.kernel(
        _hist_sc_body,
        out_type=jax.ShapeDtypeStruct((32, _HIST_WORDS), jnp.int32),
        mesh=mesh,
        compiler_params=pltpu.CompilerParams(needs_layout_passes=False),
        scratch_types=[
            pltpu.VMEM((_CH,), jnp.float32),
            pltpu.VMEM((_CH,), jnp.float32),
            pltpu.VMEM((_HIST_WORDS,), jnp.int32),
            pltpu.SemaphoreType.DMA,
            pltpu.SemaphoreType.DMA,
        ],
    )


def _hist_sc_body(dark_hbm, out_hbm, buf_a, buf_b, hist, sem_a, sem_b):
    wid = lax.axis_index("s") * 2 + lax.axis_index("c")
    image = wid // 2
    base = (wid % 2) * _HALF

    zeros16 = jnp.zeros((_LANES,), jnp.int32)

    def zbody(j, carry):
        hist[pl.ds(j * _LANES, _LANES)] = zeros16
        return carry

    lax.fori_loop(0, _NB, zbody, 0, unroll=8)

    lane_off = lax.broadcasted_iota(jnp.int32, (_LANES,), 0) * _NB
    ones16 = jnp.ones((_LANES,), jnp.int32)

    bufs = (buf_a, buf_b)
    sems = (sem_a, sem_b)
    nchunk = _HALF // _CH
    copies = [pltpu.async_copy(dark_hbm.at[image, pl.ds(base, _CH)],
                               buf_a, sem_a)]
    for ci in range(nchunk):
        if ci + 1 < nchunk:
            copies.append(pltpu.async_copy(
                dark_hbm.at[image, pl.ds(base + (ci + 1) * _CH, _CH)],
                bufs[(ci + 1) % 2], sems[(ci + 1) % 2]))
        copies[ci].wait()
        buf = bufs[ci % 2]

        def gbody(j, carry, buf=buf):
            v = buf[pl.ds(j * _LANES, _LANES)]
            b = jnp.minimum((v * float(_NB)).astype(jnp.int32), _NB - 1)
            plsc.addupdate_scatter(hist, [b + lane_off], ones16)
            return carry

        lax.fori_loop(0, _CH // _LANES, gbody, 0, unroll=8)

    pltpu.sync_copy(hist, out_hbm.at[wid])


def _crit_body(parts_ref, out_ref):
    i = pl.program_id(0)
    x = parts_ref[0]
    h = x.sum(axis=0).sum(axis=0)

    s = h
    for sh in (1, 2, 4, 8, 16, 32, 64):
        s = s + jnp.concatenate(
            [s[:, sh:], jnp.zeros((32, sh), jnp.int32)], axis=1)
    strict_lane = s - h

    rt = s[:, 0:1]
    rs = rt
    for sh in (1, 2, 4, 8, 16):
        rs = rs + jnp.concatenate(
            [rs[sh:], jnp.zeros((sh, 1), jnp.int32)], axis=0)
    row_strict = jnp.concatenate(
        [rs[1:], jnp.zeros((1, 1), jnp.int32)], axis=0)

    c_above = strict_lane + row_strict
    crit = jnp.sum((c_above >= _TOPK).astype(jnp.int32))
    out_ref[i] = crit


def _sums_body(crit_ref, img_ref, dark_ref, out_ref):
    i = pl.program_id(0)
    crit = crit_ref[i]
    d = dark_ref[0]
    b = jnp.minimum((d * float(_NB)).astype(jnp.int32), _NB - 1)
    above = b > crit
    ateq = b == crit
    img0 = img_ref[0]
    for c in range(3):
        out_ref[i, c] = jnp.sum(jnp.where(above, img0[c], 0.0))
        out_ref[i, 4 + c] = jnp.sum(jnp.where(ateq, img0[c], 0.0))
    out_ref[i, 3] = jnp.sum(above.astype(jnp.float32))
    out_ref[i, 7] = jnp.sum(ateq.astype(jnp.float32))


def _final_body(w_ref, sums_ref, img_ref, dark_ref, out_ref):
    i = pl.program_id(0)
    w = w_ref[i]
    c_above = sums_ref[i, 3]
    c_bin = sums_ref[i, 7]
    frac = (float(_TOPK) - c_above) / jnp.maximum(c_bin, 1.0)
    frac = jnp.clip(frac, 0.0, 1.0)
    d = dark_ref[0]
    t = jnp.maximum(1.0 - w * d, 0.1) + 0.001
    r = 1.0 / t
    for c in range(3):
        atm = (sums_ref[i, c] + frac * sums_ref[i, 4 + c]) * (1.0 / _TOPK)
        out_ref[0, c] = jnp.clip((img_ref[0, c] - atm) * r + atm, 0.0, 1.0)


def _dark_call(img):
    return pl.pallas_call(
        _dark_body,
        grid=(_N,),
        in_specs=[pl.BlockSpec((1, _C, _H, _W), lambda i: (i, 0, 0, 0))],
        out_specs=pl.BlockSpec((1, _H, _W), lambda i: (i, 0, 0)),
        out_shape=jax.ShapeDtypeStruct((_N, _H, _W), jnp.float32),
    )(img)


def _crit_call(parts):
    return pl.pallas_call(
        _crit_body,
        grid=(_N,),
        in_specs=[pl.BlockSpec((1, 2, _LANES, 32, 128),
                               lambda i: (i, 0, 0, 0, 0))],
        out_specs=pl.BlockSpec(memory_space=pltpu.SMEM),
        out_shape=jax.ShapeDtypeStruct((_N,), jnp.int32),
    )(parts)


def _sums_call(crit, img, dark):
    return pl.pallas_call(
        _sums_body,
        grid=(_N,),
        in_specs=[
            pl.BlockSpec(memory_space=pltpu.SMEM),
            pl.BlockSpec((1, _C, _H, _W), lambda i: (i, 0, 0, 0)),
            pl.BlockSpec((1, _H, _W), lambda i: (i, 0, 0)),
        ],
        out_specs=pl.BlockSpec(memory_space=pltpu.SMEM),
        out_shape=jax.ShapeDtypeStruct((_N, 8), jnp.float32),
    )(crit, img, dark)


def _final_call(w, sums, img, dark):
    return pl.pallas_call(
        _final_body,
        grid=(_N,),
        in_specs=[
            pl.BlockSpec(memory_space=pltpu.SMEM),
            pl.BlockSpec(memory_space=pltpu.SMEM),
            pl.BlockSpec((1, _C, _H, _W), lambda i: (i, 0, 0, 0)),
            pl.BlockSpec((1, _H, _W), lambda i: (i, 0, 0)),
        ],
        out_specs=pl.BlockSpec((1, _C, _H, _W), lambda i: (i, 0, 0, 0)),
        out_shape=jax.ShapeDtypeStruct((_N, _C, _H, _W), jnp.float32),
    )(w, sums, img, dark)


def kernel(img, w):
    dark = _dark_call(img)
    parts = _hist_call()(dark.reshape(_N, _P))
    crit = _crit_call(parts.reshape(_N, 2, _LANES, 32, 128))
    sums = _sums_call(crit, img, dark)
    return _final_call(w, sums, img, dark)
